# baseline (speedup 1.0000x reference)
import functools

import jax
import jax.numpy as jnp
from jax import lax
from jax.experimental import pallas as pl
from jax.experimental.pallas import tpu as pltpu
from jax.experimental.pallas import tpu_sc as plsc

SEQ, B, D = 200, 1024, 64
N = SEQ * B
NC, NS = 2, 16
NW = NC * NS
ROWS_W = N // NW
CHUNK = 128
NCHUNK = ROWS_W // CHUNK
GROUPS = CHUNK // 16
CT_ROWS = 24 * 4 * 7
NBUF = 2


def _prep_body(src_ref, time_ref, wd_ref, hour_ref, minute_ref, wde_ref,
               srcp_ref, ct_idx_ref, ct_tab_ref):
    srcp_ref[...] = src_ref[...] >> 1
    ct_idx_ref[...] = time_ref[...] * 7 + wd_ref[...]
    h = hour_ref[...]
    mi = minute_ref[...]
    w = wde_ref[...]
    ct_tab_ref[...] = (h[:, None, None, :] + mi[None, :, None, :]
                      + w[None, None, :, :])


_prep = pl.pallas_call(
    _prep_body,
    out_shape=(
        jax.ShapeDtypeStruct((SEQ, B), jnp.int32),
        jax.ShapeDtypeStruct((SEQ, B), jnp.int32),
        jax.ShapeDtypeStruct((24, 4, 7, D), jnp.float32),
    ),
)


def _sc_body(loc_hbm, ct_tab_hbm, dur_tab_hbm, srcp_hbm, src_hbm, ct_hbm,
             dur_hbm, out_hbm, ct_v, durt_v, srcpf_v, srcf_v, ctf_v, durf_v,
             gbuf0, gbuf1, sbuf0, sbuf1, g0, g1, s0, s1):
    wid = lax.axis_index("s") * NC + lax.axis_index("c")
    base_w = wid * ROWS_W
    gbufs, sbufs, gsems, ssems = [gbuf0, gbuf1], [sbuf0, sbuf1], [g0, g1], [s0, s1]
    pltpu.sync_copy(ct_tab_hbm, ct_v)
    pltpu.sync_copy(dur_tab_hbm, durt_v)
    pltpu.sync_copy(srcp_hbm.at[pl.ds(base_w, ROWS_W)], srcpf_v)
    pltpu.sync_copy(src_hbm.at[pl.ds(base_w, ROWS_W)], srcf_v)
    pltpu.sync_copy(ct_hbm.at[pl.ds(base_w, ROWS_W)], ctf_v)
    pltpu.sync_copy(dur_hbm.at[pl.ds(base_w, ROWS_W)], durf_v)
    col_iota = lax.iota(jnp.int32, 16)

    def start_gather(c, b):
        pltpu.async_copy(loc_hbm.at[srcpf_v.at[pl.ds(c * CHUNK, CHUNK)]],
                         gbufs[b], gsems[b])

    def wait_gather(c, b):
        pltpu.make_async_copy(loc_hbm.at[srcpf_v.at[pl.ds(c * CHUNK, CHUNK)]],
                              gbufs[b], gsems[b]).wait()

    for b in range(NBUF):
        start_gather(b, b)

    def outer(i, carry):
        c0 = i * NBUF
        for b in range(NBUF):
            c = c0 + b
            wait_gather(c, b)

            @pl.when(c >= NBUF)
            def _():
                pltpu.make_async_copy(
                    sbufs[b], out_hbm.at[pl.ds(0, CHUNK * D)], ssems[b]).wait()

            @plsc.parallel_loop(0, GROUPS, unroll=2)
            def group_body(g):
                gb = c * CHUNK + g * 16
                ct16 = ctf_v[pl.ds(gb, 16)] * D
                dur16 = durf_v[pl.ds(gb, 16)] * D
                src16 = srcf_v[pl.ds(gb, 16)]
                off16 = (src16 & 1) * D
                keep16 = jnp.where(src16 == 0, 0.0, 1.0)
                for j in range(16):
                    r = g * 16 + j
                    ct_r, dur_r = ct16[j], dur16[j]
                    keep = keep16[j]
                    lcol = col_iota + off16[j]
                    rsp = jnp.full((16,), r, jnp.int32)
                    for k in range(D // 16):
                        a = ct_v[pl.ds(ct_r + k * 16, 16)]
                        t = durt_v[pl.ds(dur_r + k * 16, 16)]
                        l = plsc.load_gather(gbufs[b], [rsp, lcol + k * 16])
                        sbufs[b][pl.ds(r * D + k * 16, 16)] = a + t + l * keep

            pltpu.async_copy(
                sbufs[b],
                out_hbm.at[pl.ds((base_w + c * CHUNK) * D, CHUNK * D)],
                ssems[b])

            @pl.when(c + NBUF < NCHUNK)
            def _():
                start_gather(c + NBUF, b)
        return carry

    lax.fori_loop(0, NCHUNK // NBUF, outer, 0)
    for b in range(NBUF):
        pltpu.make_async_copy(
            sbufs[b], out_hbm.at[pl.ds(0, CHUNK * D)], ssems[b]).wait()


_sc_embed = functools.partial(
    pl.kernel,
    out_type=jax.ShapeDtypeStruct((N * D,), jnp.float32),
    mesh=plsc.VectorSubcoreMesh(core_axis_name="c", subcore_axis_name="s"),
    compiler_params=pltpu.CompilerParams(needs_layout_passes=False,
                                         use_tc_tiling_on_sc=True),
    scratch_types=[
        pltpu.VMEM((CT_ROWS * D,), jnp.float32),
        pltpu.VMEM((96 * D,), jnp.float32),
        pltpu.VMEM((ROWS_W,), jnp.int32),
        pltpu.VMEM((ROWS_W,), jnp.int32),
        pltpu.VMEM((ROWS_W,), jnp.int32),
        pltpu.VMEM((ROWS_W,), jnp.int32),
        pltpu.VMEM((CHUNK, 2 * D), jnp.float32),
        pltpu.VMEM((CHUNK, 2 * D), jnp.float32),
        pltpu.VMEM((CHUNK * D,), jnp.float32),
        pltpu.VMEM((CHUNK * D,), jnp.float32),
        pltpu.SemaphoreType.DMA,
        pltpu.SemaphoreType.DMA,
        pltpu.SemaphoreType.DMA,
        pltpu.SemaphoreType.DMA,
    ],
)(_sc_body)


def kernel(src, time, weekday, duration, emb_loc, minute_embed, hour_embed,
           weekday_embed, emb_duration):
    src = src.astype(jnp.int32)
    srcp, ct_idx, ct_tab4 = _prep(src, time.astype(jnp.int32),
                                  weekday.astype(jnp.int32),
                                  hour_embed, minute_embed, weekday_embed)
    out1 = _sc_embed(emb_loc.reshape(1000000 // 2, 2 * D),
                     ct_tab4.reshape(CT_ROWS * D),
                     emb_duration.reshape(96 * D),
                     srcp.reshape(N),
                     src.reshape(N),
                     ct_idx.reshape(N),
                     duration.reshape(N).astype(jnp.int32))
    return out1.reshape(SEQ, B, D)


# speedup vs baseline: 1.0798x; 1.0798x over previous
import functools

import jax
import jax.numpy as jnp
from jax import lax
from jax.experimental import pallas as pl
from jax.experimental.pallas import tpu as pltpu
from jax.experimental.pallas import tpu_sc as plsc

SEQ, B, D = 200, 1024, 64
N = SEQ * B
NC, NS = 2, 16
NW = NC * NS
ROWS_W = N // NW
CHUNK = 128
NCHUNK = ROWS_W // CHUNK
GROUPS = CHUNK // 16
CT_ROWS = 24 * 4 * 7
NBUF = 2


def _prep_body(src_ref, time_ref, wd_ref, hour_ref, minute_ref, wde_ref,
               ct_idx_ref, ct_tab_ref):
    ct_idx_ref[...] = time_ref[...] * 7 + wd_ref[...]
    h = hour_ref[...]
    mi = minute_ref[...]
    w = wde_ref[...]
    ct_tab_ref[...] = (h[:, None, None, :] + mi[None, :, None, :]
                      + w[None, None, :, :])


_prep = pl.pallas_call(
    _prep_body,
    out_shape=(
        jax.ShapeDtypeStruct((SEQ, B), jnp.int32),
        jax.ShapeDtypeStruct((24, 4, 7, D), jnp.float32),
    ),
)


def _sc_body(loc_hbm, ct_tab_hbm, dur_tab_hbm, src_hbm, ct_hbm,
             dur_hbm, out_hbm, ct_v, durt_v, srcf_v, ctf_v, durf_v,
             gbuf0, gbuf1, sbuf0, sbuf1, g0, g1, s0, s1):
    wid = lax.axis_index("s") * NC + lax.axis_index("c")
    base_w = wid * ROWS_W
    gbufs, sbufs, gsems, ssems = [gbuf0, gbuf1], [sbuf0, sbuf1], [g0, g1], [s0, s1]
    pltpu.sync_copy(ct_tab_hbm, ct_v)
    pltpu.sync_copy(dur_tab_hbm, durt_v)
    pltpu.sync_copy(src_hbm.at[pl.ds(base_w, ROWS_W)], srcf_v)
    pltpu.sync_copy(ct_hbm.at[pl.ds(base_w, ROWS_W)], ctf_v)
    pltpu.sync_copy(dur_hbm.at[pl.ds(base_w, ROWS_W)], durf_v)
    col_iota = lax.iota(jnp.int32, 16)

    def start_gather(c, b):
        pltpu.async_copy(loc_hbm.at[srcf_v.at[pl.ds(c * CHUNK, CHUNK)]],
                         gbufs[b], gsems[b])

    def wait_gather(c, b):
        pltpu.make_async_copy(loc_hbm.at[srcf_v.at[pl.ds(c * CHUNK, CHUNK)]],
                              gbufs[b], gsems[b]).wait()

    for b in range(NBUF):
        start_gather(b, b)

    def outer(i, carry):
        c0 = i * NBUF
        for b in range(NBUF):
            c = c0 + b
            wait_gather(c, b)

            @pl.when(c >= NBUF)
            def _():
                pltpu.make_async_copy(
                    sbufs[b], out_hbm.at[pl.ds(0, CHUNK * D)], ssems[b]).wait()

            @plsc.parallel_loop(0, GROUPS, unroll=2)
            def group_body(g):
                gb = c * CHUNK + g * 16
                ct16 = ctf_v[pl.ds(gb, 16)] * D
                dur16 = durf_v[pl.ds(gb, 16)] * D
                src16 = srcf_v[pl.ds(gb, 16)]
                keep16 = jnp.where(src16 == 0, 0.0, 1.0)
                for j in range(16):
                    r = g * 16 + j
                    ct_r, dur_r = ct16[j], dur16[j]
                    keep = keep16[j]
                    lcol = col_iota
                    rsp = jnp.full((16,), r, jnp.int32)
                    for k in range(D // 16):
                        a = ct_v[pl.ds(ct_r + k * 16, 16)]
                        t = durt_v[pl.ds(dur_r + k * 16, 16)]
                        l = plsc.load_gather(gbufs[b], [rsp, lcol + k * 16])
                        sbufs[b][pl.ds(r * D + k * 16, 16)] = a + t + l * keep

            pltpu.async_copy(
                sbufs[b],
                out_hbm.at[pl.ds((base_w + c * CHUNK) * D, CHUNK * D)],
                ssems[b])

            @pl.when(c + NBUF < NCHUNK)
            def _():
                start_gather(c + NBUF, b)
        return carry

    lax.fori_loop(0, NCHUNK // NBUF, outer, 0)
    for b in range(NBUF):
        pltpu.make_async_copy(
            sbufs[b], out_hbm.at[pl.ds(0, CHUNK * D)], ssems[b]).wait()


_sc_embed = functools.partial(
    pl.kernel,
    out_type=jax.ShapeDtypeStruct((N * D,), jnp.float32),
    mesh=plsc.VectorSubcoreMesh(core_axis_name="c", subcore_axis_name="s"),
    compiler_params=pltpu.CompilerParams(needs_layout_passes=False,
                                         use_tc_tiling_on_sc=True),
    scratch_types=[
        pltpu.VMEM((CT_ROWS * D,), jnp.float32),
        pltpu.VMEM((96 * D,), jnp.float32),
        pltpu.VMEM((ROWS_W,), jnp.int32),
        pltpu.VMEM((ROWS_W,), jnp.int32),
        pltpu.VMEM((ROWS_W,), jnp.int32),
        pltpu.VMEM((CHUNK, 2 * D), jnp.float32),
        pltpu.VMEM((CHUNK, 2 * D), jnp.float32),
        pltpu.VMEM((CHUNK * D,), jnp.float32),
        pltpu.VMEM((CHUNK * D,), jnp.float32),
        pltpu.SemaphoreType.DMA,
        pltpu.SemaphoreType.DMA,
        pltpu.SemaphoreType.DMA,
        pltpu.SemaphoreType.DMA,
    ],
)(_sc_body)


def kernel(src, time, weekday, duration, emb_loc, minute_embed, hour_embed,
           weekday_embed, emb_duration):
    src = src.astype(jnp.int32)
    ct_idx, ct_tab4 = _prep(src, time.astype(jnp.int32),
                            weekday.astype(jnp.int32),
                            hour_embed, minute_embed, weekday_embed)
    out1 = _sc_embed(jnp.pad(emb_loc, ((0, 0), (0, D))),
                     ct_tab4.reshape(CT_ROWS * D),
                     emb_duration.reshape(96 * D),
                     src.reshape(N),
                     ct_idx.reshape(N),
                     duration.reshape(N).astype(jnp.int32))
    return out1.reshape(SEQ, B, D)


# speedup vs baseline: 1.1613x; 1.0755x over previous
import functools

import jax
import jax.numpy as jnp
from jax import lax
from jax.experimental import pallas as pl
from jax.experimental.pallas import tpu as pltpu
from jax.experimental.pallas import tpu_sc as plsc

SEQ, B, D = 200, 1024, 64
N = SEQ * B
NC, NS = 2, 16
NW = NC * NS
ROWS_W = N // NW
CHUNK = 128
NCHUNK = ROWS_W // CHUNK
GROUPS = CHUNK // 16
CT_ROWS = 24 * 4 * 7
NBUF = 2


def _prep_body(src_ref, time_ref, wd_ref, hour_ref, minute_ref, wde_ref,
               ct_idx_ref, ct_tab_ref):
    ct_idx_ref[...] = time_ref[...] * 7 + wd_ref[...]
    h = hour_ref[...]
    mi = minute_ref[...]
    w = wde_ref[...]
    ct_tab_ref[...] = (h[:, None, None, :] + mi[None, :, None, :]
                      + w[None, None, :, :])


_prep = pl.pallas_call(
    _prep_body,
    out_shape=(
        jax.ShapeDtypeStruct((SEQ, B), jnp.int32),
        jax.ShapeDtypeStruct((24, 4, 7, D), jnp.float32),
    ),
)


WCOLS = 2048


def _widen_body(x_ref, o_ref):
    o_ref[...] = jnp.pad(x_ref[...].T, ((0, 0), (0, D)))


_widen = pl.pallas_call(
    _widen_body,
    grid=(pl.cdiv(1000000, WCOLS),),
    in_specs=[pl.BlockSpec((D, WCOLS), lambda j: (0, j))],
    out_specs=pl.BlockSpec((WCOLS, 2 * D), lambda j: (j, 0)),
    out_shape=jax.ShapeDtypeStruct((1000000, 2 * D), jnp.float32),
)


def _sc_body(loc_hbm, ct_tab_hbm, dur_tab_hbm, src_hbm, ct_hbm,
             dur_hbm, out_hbm, ct_v, durt_v, srcf_v, ctf_v, durf_v,
             gbuf0, gbuf1, sbuf0, sbuf1, g0, g1, s0, s1):
    wid = lax.axis_index("s") * NC + lax.axis_index("c")
    base_w = wid * ROWS_W
    gbufs, sbufs, gsems, ssems = [gbuf0, gbuf1], [sbuf0, sbuf1], [g0, g1], [s0, s1]
    pltpu.sync_copy(ct_tab_hbm, ct_v)
    pltpu.sync_copy(dur_tab_hbm, durt_v)
    pltpu.sync_copy(src_hbm.at[pl.ds(base_w, ROWS_W)], srcf_v)
    pltpu.sync_copy(ct_hbm.at[pl.ds(base_w, ROWS_W)], ctf_v)
    pltpu.sync_copy(dur_hbm.at[pl.ds(base_w, ROWS_W)], durf_v)
    col_iota = lax.iota(jnp.int32, 16)

    def start_gather(c, b):
        pltpu.async_copy(loc_hbm.at[srcf_v.at[pl.ds(c * CHUNK, CHUNK)]],
                         gbufs[b], gsems[b])

    def wait_gather(c, b):
        pltpu.make_async_copy(loc_hbm.at[srcf_v.at[pl.ds(c * CHUNK, CHUNK)]],
                              gbufs[b], gsems[b]).wait()

    for b in range(NBUF):
        start_gather(b, b)

    def outer(i, carry):
        c0 = i * NBUF
        for b in range(NBUF):
            c = c0 + b
            wait_gather(c, b)

            @pl.when(c >= NBUF)
            def _():
                pltpu.make_async_copy(
                    sbufs[b], out_hbm.at[pl.ds(0, CHUNK * D)], ssems[b]).wait()

            @plsc.parallel_loop(0, GROUPS, unroll=2)
            def group_body(g):
                gb = c * CHUNK + g * 16
                ct16 = ctf_v[pl.ds(gb, 16)] * D
                dur16 = durf_v[pl.ds(gb, 16)] * D
                src16 = srcf_v[pl.ds(gb, 16)]
                keep16 = jnp.where(src16 == 0, 0.0, 1.0)
                for j in range(16):
                    r = g * 16 + j
                    ct_r, dur_r = ct16[j], dur16[j]
                    keep = keep16[j]
                    lcol = col_iota
                    rsp = jnp.full((16,), r, jnp.int32)
                    for k in range(D // 16):
                        a = ct_v[pl.ds(ct_r + k * 16, 16)]
                        t = durt_v[pl.ds(dur_r + k * 16, 16)]
                        l = plsc.load_gather(gbufs[b], [rsp, lcol + k * 16])
                        sbufs[b][pl.ds(r * D + k * 16, 16)] = a + t + l * keep

            pltpu.async_copy(
                sbufs[b],
                out_hbm.at[pl.ds((base_w + c * CHUNK) * D, CHUNK * D)],
                ssems[b])

            @pl.when(c + NBUF < NCHUNK)
            def _():
                start_gather(c + NBUF, b)
        return carry

    lax.fori_loop(0, NCHUNK // NBUF, outer, 0)
    for b in range(NBUF):
        pltpu.make_async_copy(
            sbufs[b], out_hbm.at[pl.ds(0, CHUNK * D)], ssems[b]).wait()


_sc_embed = functools.partial(
    pl.kernel,
    out_type=jax.ShapeDtypeStruct((N * D,), jnp.float32),
    mesh=plsc.VectorSubcoreMesh(core_axis_name="c", subcore_axis_name="s"),
    compiler_params=pltpu.CompilerParams(needs_layout_passes=False,
                                         use_tc_tiling_on_sc=True),
    scratch_types=[
        pltpu.VMEM((CT_ROWS * D,), jnp.float32),
        pltpu.VMEM((96 * D,), jnp.float32),
        pltpu.VMEM((ROWS_W,), jnp.int32),
        pltpu.VMEM((ROWS_W,), jnp.int32),
        pltpu.VMEM((ROWS_W,), jnp.int32),
        pltpu.VMEM((CHUNK, 2 * D), jnp.float32),
        pltpu.VMEM((CHUNK, 2 * D), jnp.float32),
        pltpu.VMEM((CHUNK * D,), jnp.float32),
        pltpu.VMEM((CHUNK * D,), jnp.float32),
        pltpu.SemaphoreType.DMA,
        pltpu.SemaphoreType.DMA,
        pltpu.SemaphoreType.DMA,
        pltpu.SemaphoreType.DMA,
    ],
)(_sc_body)


def kernel(src, time, weekday, duration, emb_loc, minute_embed, hour_embed,
           weekday_embed, emb_duration):
    src = src.astype(jnp.int32)
    ct_idx, ct_tab4 = _prep(src, time.astype(jnp.int32),
                            weekday.astype(jnp.int32),
                            hour_embed, minute_embed, weekday_embed)
    out1 = _sc_embed(_widen(emb_loc.T),
                     ct_tab4.reshape(CT_ROWS * D),
                     emb_duration.reshape(96 * D),
                     src.reshape(N),
                     ct_idx.reshape(N),
                     duration.reshape(N).astype(jnp.int32))
    return out1.reshape(SEQ, B, D)
